# speedup vs baseline: 3.0537x; 3.0537x over previous
import functools

import jax
import jax.numpy as jnp
from jax import lax
from jax.experimental import pallas as pl
from jax.experimental.pallas import tpu as pltpu
from jax.experimental.pallas import tpu_sc as plsc

N_ROWS = 16384 * 50
DIM = 128
NW = 32
ROWS_PER_W = N_ROWS // NW
CHUNK = 128
N_CHUNKS = ROWS_PER_W // CHUNK


def _make_kernel():
    mesh = plsc.VectorSubcoreMesh(core_axis_name="c", subcore_axis_name="s")

    @functools.partial(
        pl.kernel,
        out_type=jax.ShapeDtypeStruct((N_ROWS, DIM), jnp.float32),
        mesh=mesh,
        scratch_types=[
            pltpu.VMEM((N_CHUNKS, CHUNK), jnp.int32),
            pltpu.VMEM((CHUNK, DIM), jnp.float32),
            pltpu.SemaphoreType.DMA,
            pltpu.SemaphoreType.DMA,
        ],
    )
    def gather_kernel(idx_hbm, table_hbm, out_hbm, idx_v, rows_v, gsem, ssem):
        wid = lax.axis_index("s") * 2 + lax.axis_index("c")
        base = wid * ROWS_PER_W
        pltpu.sync_copy(idx_hbm.at[wid], idx_v)

        def body(j, carry):
            pltpu.async_copy(table_hbm.at[idx_v.at[j]], rows_v, gsem).wait()
            pltpu.async_copy(
                rows_v, out_hbm.at[pl.ds(base + j * CHUNK, CHUNK)], ssem
            ).wait()
            return carry

        lax.fori_loop(0, N_CHUNKS, body, 0)

    return gather_kernel


_gather = _make_kernel()


@jax.jit
def kernel(y_true, proxies):
    idx = y_true.astype(jnp.int32).reshape(NW, N_CHUNKS, CHUNK)
    out = _gather(idx, proxies)
    return out.reshape(16384, 50, DIM)


# speedup vs baseline: 3.4382x; 1.1259x over previous
import functools

import jax
import jax.numpy as jnp
from jax import lax
from jax.experimental import pallas as pl
from jax.experimental.pallas import tpu as pltpu
from jax.experimental.pallas import tpu_sc as plsc

N_ROWS = 16384 * 50
DIM = 128
NW = 32
ROWS_PER_W = N_ROWS // NW
CHUNK = 128
N_CHUNKS = ROWS_PER_W // CHUNK
NBUF = 4
N_GROUPS = N_CHUNKS // NBUF


def _make_kernel():
    mesh = plsc.VectorSubcoreMesh(core_axis_name="c", subcore_axis_name="s")

    @functools.partial(
        pl.kernel,
        out_type=jax.ShapeDtypeStruct((N_ROWS, DIM), jnp.float32),
        mesh=mesh,
        scratch_types=[
            pltpu.VMEM((N_CHUNKS, CHUNK), jnp.int32),
            [pltpu.VMEM((CHUNK, DIM), jnp.float32) for _ in range(NBUF)],
            [pltpu.SemaphoreType.DMA for _ in range(NBUF)],
            [pltpu.SemaphoreType.DMA for _ in range(NBUF)],
        ],
    )
    def gather_kernel(idx_hbm, table_hbm, out_hbm, idx_v, rows, gsems, ssems):
        wid = lax.axis_index("s") * 2 + lax.axis_index("c")
        base = wid * ROWS_PER_W
        pltpu.sync_copy(idx_hbm.at[wid], idx_v)

        def start_gather(b, j):
            pltpu.async_copy(table_hbm.at[idx_v.at[j]], rows[b], gsems[b])

        def wait_gather(b):
            pltpu.make_async_copy(table_hbm.at[idx_v.at[0]], rows[b], gsems[b]).wait()

        def start_store(b, j):
            pltpu.async_copy(
                rows[b], out_hbm.at[pl.ds(base + j * CHUNK, CHUNK)], ssems[b]
            )

        def wait_store(b):
            pltpu.make_async_copy(
                rows[b], out_hbm.at[pl.ds(base, CHUNK)], ssems[b]
            ).wait()

        for b in range(NBUF):
            start_gather(b, b)

        def body(g, carry):
            j0 = g * NBUF
            for b in range(NBUF):
                wait_gather(b)
                start_store(b, j0 + b)
            for b in range(NBUF):
                wait_store(b)
                start_gather(b, j0 + NBUF + b)
            return carry

        lax.fori_loop(0, N_GROUPS - 1, body, 0)

        j0 = (N_GROUPS - 1) * NBUF
        for b in range(NBUF):
            wait_gather(b)
            start_store(b, j0 + b)
        for b in range(NBUF):
            wait_store(b)

    return gather_kernel


_gather = _make_kernel()


@jax.jit
def kernel(y_true, proxies):
    idx = y_true.astype(jnp.int32).reshape(NW, N_CHUNKS, CHUNK)
    out = _gather(idx, proxies)
    return out.reshape(16384, 50, DIM)


# speedup vs baseline: 6.2950x; 1.8309x over previous
import functools

import jax
import jax.numpy as jnp
from jax import lax
from jax.experimental import pallas as pl
from jax.experimental.pallas import tpu as pltpu
from jax.experimental.pallas import tpu_sc as plsc

BATCH = 16384
HIST = 50
DIM = 128
NW = 32
BATCH_PER_W = BATCH // NW
CHUNK_B = 2
CHUNK = CHUNK_B * HIST
N_CHUNKS = BATCH_PER_W // CHUNK_B
NBUF = 4
N_GROUPS = N_CHUNKS // NBUF


def _make_kernel():
    mesh = plsc.VectorSubcoreMesh(core_axis_name="c", subcore_axis_name="s")

    @functools.partial(
        pl.kernel,
        out_type=jax.ShapeDtypeStruct((BATCH, HIST, DIM), jnp.float32),
        mesh=mesh,
        scratch_types=[
            pltpu.VMEM((N_CHUNKS, CHUNK), jnp.int32),
            [pltpu.VMEM((CHUNK, DIM), jnp.float32) for _ in range(NBUF)],
            [pltpu.SemaphoreType.DMA for _ in range(NBUF)],
            [pltpu.SemaphoreType.DMA for _ in range(NBUF)],
        ],
    )
    def gather_kernel(idx_hbm, table_hbm, out_hbm, idx_v, rows, gsems, ssems):
        wid = lax.axis_index("s") * 2 + lax.axis_index("c")
        base_b = wid * BATCH_PER_W
        pltpu.sync_copy(idx_hbm.at[wid], idx_v)

        def start_gather(b, j):
            pltpu.async_copy(table_hbm.at[idx_v.at[j]], rows[b], gsems[b])

        def wait_gather(b):
            pltpu.make_async_copy(table_hbm.at[idx_v.at[0]], rows[b], gsems[b]).wait()

        def start_store(b, j):
            b0 = base_b + j * CHUNK_B
            pltpu.async_copy(rows[b].at[pl.ds(0, HIST)], out_hbm.at[b0], ssems[b])
            pltpu.async_copy(
                rows[b].at[pl.ds(HIST, HIST)], out_hbm.at[b0 + 1], ssems[b]
            )

        def wait_store(b):
            pltpu.make_async_copy(
                rows[b].at[pl.ds(0, HIST)], out_hbm.at[0], ssems[b]
            ).wait()
            pltpu.make_async_copy(
                rows[b].at[pl.ds(0, HIST)], out_hbm.at[0], ssems[b]
            ).wait()

        for b in range(NBUF):
            start_gather(b, b)

        def body(g, carry):
            j0 = g * NBUF
            for b in range(NBUF):
                wait_gather(b)
                start_store(b, j0 + b)
            for b in range(NBUF):
                wait_store(b)
                start_gather(b, j0 + NBUF + b)
            return carry

        lax.fori_loop(0, N_GROUPS - 1, body, 0)

        j0 = (N_GROUPS - 1) * NBUF
        for b in range(NBUF):
            wait_gather(b)
            start_store(b, j0 + b)
        for b in range(NBUF):
            wait_store(b)

    return gather_kernel


_gather = _make_kernel()


@jax.jit
def kernel(y_true, proxies):
    idx = y_true.astype(jnp.int32).reshape(NW, N_CHUNKS, CHUNK)
    return _gather(idx, proxies)
